# baseline (speedup 1.0000x reference)
import functools

import jax
import jax.numpy as jnp
from jax import lax
from jax.experimental import pallas as pl
from jax.experimental.pallas import tpu as pltpu
from jax.experimental.pallas import tpu_sc as plsc

_NB = 4
_NP = 1722
_NM = 12
_HID = 552
_NPP = 1792
_NK = 14
_JOBS = _NB * _NM * _NK
_NA = 12
_NBT = 20
_AJOBS = _JOBS // _NA
_BJOBS = -(-_JOBS // _NBT)
_JPB = _NM * _NK
_RPAD = (_NPP - _NP) * 48


def _bucket(x):
    e = lax.shift_right_logical(plsc.bitcast(x.astype(jnp.float32), jnp.int32), 23)
    return jnp.maximum(e - 126, 0)


def _make_sc_call():
    mesh = plsc.VectorSubcoreMesh(core_axis_name="c", subcore_axis_name="s")

    @functools.partial(
        pl.kernel,
        mesh=mesh,
        compiler_params=pltpu.CompilerParams(needs_layout_passes=False),
        out_type=[
            jax.ShapeDtypeStruct((_NB, _NM, _HID, _NPP), jnp.float32),
            jax.ShapeDtypeStruct((_NA, _AJOBS * 128), jnp.int32),
        ],
        scratch_types=[
            pltpu.VMEM((500, 128), jnp.float32),
            pltpu.VMEM((40, 20), jnp.float32),
            pltpu.VMEM((256,), jnp.int32),
            pltpu.VMEM((2 * 6144,), jnp.int32),
            pltpu.VMEM((144, 128), jnp.float32),
            pltpu.VMEM((152, 128), jnp.float32),
            pltpu.VMEM((_AJOBS * 128,), jnp.int32),
            pltpu.SemaphoreType.DMA((2,)),
            pltpu.SemaphoreType.DMA,
        ],
    )
    def sc_kernel(rel, pinfo, graph, ctx, discat, outf, outm,
                  table_v, discat_v, pinfo_v, idx_v, bufx_v, bufy_v,
                  mask_v, sem, sem2):
        cid = lax.axis_index("c")
        sid = lax.axis_index("s")
        wid = sid * 2 + cid
        lane = lax.iota(jnp.int32, 16)
        lane48 = lane * 48

        def slab_src(b, k):
            return rel.at[pl.ds((b * _NP + k * 128) * 48, 6144)]

        def load_ids(slot, m, cl):
            koff = slot * 6144 + cl * 48 + m * 4
            i0 = jnp.clip(plsc.load_gather(idx_v, [koff]), 0, 255)
            i1 = jnp.clip(plsc.load_gather(idx_v, [koff + 1]), 0, 255)
            i2 = jnp.clip(plsc.load_gather(idx_v, [koff + 2]), 0, 255)
            i3 = jnp.clip(plsc.load_gather(idx_v, [koff + 3]), 0, 255)
            return i0, i1, i2, i3

        @pl.when(wid < _NA)
        def _a_role():
            ta = wid
            b0 = ta // (_NA // _NB)
            pltpu.sync_copy(graph.at[b0], table_v.at[pl.ds(0, 256)])
            pltpu.sync_copy(discat, discat_v)
            pltpu.sync_copy(pinfo.at[b0], pinfo_v)
            r0 = lax.rem(ta * _AJOBS, _JPB)
            pltpu.sync_copy(slab_src(b0, lax.rem(r0, _NK)),
                            idx_v.at[pl.ds(0, 6144)])

            def unit(u, carry):
                i = u // 2
                half = lax.rem(u, 2)
                js = lax.rem(i, 2)
                r = lax.rem(ta * _AJOBS + i, _JPB)
                m = r // _NK
                k = lax.rem(r, _NK)

                @pl.when(jnp.logical_and(half == 0, i >= 1))
                def _ws():
                    pltpu.make_async_copy(
                        slab_src(b0, k),
                        idx_v.at[pl.ds(js * 6144, 6144)], sem2).wait()

                @pl.when(jnp.logical_and(half == 0, i + 1 < _AJOBS))
                def _ps():
                    r1 = lax.rem(ta * _AJOBS + i + 1, _JPB)
                    pltpu.async_copy(
                        slab_src(b0, lax.rem(r1, _NK)),
                        idx_v.at[pl.ds(lax.rem(i + 1, 2) * 6144, 6144)], sem2)

                @pl.when(jnp.logical_and(u >= 2, half == 0))
                def _wx():
                    pltpu.make_async_copy(
                        bufx_v,
                        outf.at[b0, 0, pl.ds(0, 144), pl.ds(0, 128)],
                        sem.at[0]).wait()

                @pl.when(jnp.logical_and(u >= 2, half == 1))
                def _wy():
                    pltpu.make_async_copy(
                        bufy_v,
                        outf.at[b0, 0, pl.ds(144, 152), pl.ds(0, 128)],
                        sem.at[1]).wait()

                def subchunk(c, cc):
                    cl = c * 16 + lane
                    i0, i1, i2, i3 = load_ids(js, m, cl)
                    isel = jnp.where(jnp.broadcast_to(m < 4, (16,)), i2, i3)
                    ssum = i0 + i1 + i2 + i3
                    mask_f = jnp.where(ssum > 0, 1.0, 0.0).astype(jnp.float32)

                    @pl.when(half == 0)
                    def _h0():
                        plsc.store_scatter(mask_v, [i * 128 + cl], ssum)

                        @plsc.parallel_loop(0, 128, unroll=8)
                        def _g0(cu):
                            cuv = jnp.broadcast_to(cu, (16,))
                            v = plsc.load_gather(table_v, [i0, cuv]) * mask_f
                            plsc.store_scatter(bufx_v, [cuv, cl], v)

                        @plsc.parallel_loop(0, 16, unroll=4)
                        def _g1(cu):
                            cuv = jnp.broadcast_to(cu, (16,))
                            v = plsc.load_gather(table_v, [isel, cuv]) * mask_f
                            plsc.store_scatter(bufx_v, [cuv + 128, cl], v)

                    @pl.when(half == 1)
                    def _h1():
                        a0 = plsc.load_gather(pinfo_v, [i0])
                        asel = plsc.load_gather(pinfo_v, [isel])
                        delta = a0 - asel
                        xeff = jnp.clip(
                            jnp.where(delta < 0, delta + 512, delta), 0, 511)
                        d = _bucket(xeff)
                        di = jnp.where(delta < 0, 10 - d, 10 + d)
                        di2 = _bucket(di) + 30

                        @plsc.parallel_loop(0, 112, unroll=8)
                        def _g2(cu):
                            cuv = jnp.broadcast_to(cu, (16,))
                            v = plsc.load_gather(
                                table_v, [isel, cuv + 16]) * mask_f
                            plsc.store_scatter(bufy_v, [cuv, cl], v)

                        @plsc.parallel_loop(0, 20, unroll=4)
                        def _g3(cu):
                            cuv = jnp.broadcast_to(cu, (16,))
                            e0 = plsc.load_gather(discat_v, [di, cuv]) * mask_f
                            plsc.store_scatter(bufy_v, [cuv + 112, cl], e0)
                            e1 = plsc.load_gather(discat_v, [di2, cuv]) * mask_f
                            plsc.store_scatter(bufy_v, [cuv + 132, cl], e1)

                    return cc
                lax.fori_loop(0, 8, subchunk, 0)

                @pl.when(half == 0)
                def _dx():
                    pltpu.async_copy(
                        bufx_v,
                        outf.at[b0, m, pl.ds(0, 144), pl.ds(k * 128, 128)],
                        sem.at[0])

                @pl.when(half == 1)
                def _dy():
                    pltpu.async_copy(
                        bufy_v,
                        outf.at[b0, m, pl.ds(144, 152), pl.ds(k * 128, 128)],
                        sem.at[1])
                return carry

            lax.fori_loop(0, 2 * _AJOBS, unit, 0)
            pltpu.make_async_copy(
                bufx_v, outf.at[b0, 0, pl.ds(0, 144), pl.ds(0, 128)],
                sem.at[0]).wait()
            pltpu.make_async_copy(
                bufy_v, outf.at[b0, 0, pl.ds(144, 152), pl.ds(0, 128)],
                sem.at[1]).wait()
            pltpu.sync_copy(mask_v, outm.at[ta])

        @pl.when(wid >= _NA)
        def _b_role():
            tb = wid - _NA
            b0 = (tb * _BJOBS) // _JPB
            pltpu.sync_copy(ctx.at[b0], table_v)
            pltpu.sync_copy(pinfo.at[b0], pinfo_v)
            r0 = lax.rem(tb * _BJOBS, _JPB)
            pltpu.sync_copy(slab_src(b0, lax.rem(r0, _NK)),
                            idx_v.at[pl.ds(0, 6144)])

            def unit(u, bcur):
                i = u // 2
                half = lax.rem(u, 2)
                js = lax.rem(i, 2)
                j = jnp.minimum(tb * _BJOBS + i, _JOBS - 1)
                bj = j // _JPB
                r = lax.rem(j, _JPB)
                m = r // _NK
                k = lax.rem(r, _NK)

                @pl.when(bj != bcur)
                def _reload():
                    pltpu.sync_copy(ctx.at[bj], table_v)
                    pltpu.sync_copy(pinfo.at[bj], pinfo_v)

                @pl.when(jnp.logical_and(half == 0, i >= 1))
                def _ws():
                    pltpu.make_async_copy(
                        slab_src(bj, k),
                        idx_v.at[pl.ds(js * 6144, 6144)], sem2).wait()

                @pl.when(jnp.logical_and(half == 0, i + 1 < _BJOBS))
                def _ps():
                    j1 = jnp.minimum(tb * _BJOBS + i + 1, _JOBS - 1)
                    r1 = lax.rem(j1, _JPB)
                    pltpu.async_copy(
                        slab_src(j1 // _JPB, lax.rem(r1, _NK)),
                        idx_v.at[pl.ds(lax.rem(i + 1, 2) * 6144, 6144)], sem2)

                @pl.when(jnp.logical_and(u >= 2, half == 0))
                def _wx():
                    pltpu.make_async_copy(
                        bufx_v.at[pl.ds(0, 128)],
                        outf.at[bj, 0, pl.ds(296, 128), pl.ds(0, 128)],
                        sem.at[0]).wait()

                @pl.when(jnp.logical_and(u >= 2, half == 1))
                def _wy():
                    pltpu.make_async_copy(
                        bufy_v.at[pl.ds(0, 128)],
                        outf.at[bj, 0, pl.ds(424, 128), pl.ds(0, 128)],
                        sem.at[1]).wait()

                def subchunk(c, cc):
                    cl = c * 16 + lane
                    i0, i1, i2, i3 = load_ids(js, m, cl)
                    ssum = i0 + i1 + i2 + i3
                    mask_f = jnp.where(ssum > 0, 1.0, 0.0).astype(jnp.float32)
                    wlog = jnp.where(jnp.broadcast_to(m >= 8, (16,)),
                                     1.0, 0.0).astype(jnp.float32)
                    a0 = jnp.clip(plsc.load_gather(pinfo_v, [i0]), 0, 499)
                    a1 = jnp.clip(plsc.load_gather(pinfo_v, [i1]), 0, 499)
                    a2 = jnp.clip(plsc.load_gather(pinfo_v, [i2]), 0, 499)
                    a3 = jnp.clip(plsc.load_gather(pinfo_v, [i3]), 0, 499)

                    @pl.when(half == 0)
                    def _h0():
                        @plsc.parallel_loop(0, 128, unroll=8)
                        def _c0(cu):
                            cuv = jnp.broadcast_to(cu, (16,))
                            u0 = plsc.load_gather(table_v, [a0, cuv])
                            u1 = plsc.load_gather(table_v, [a1, cuv])
                            plsc.store_scatter(bufx_v, [cuv, cl],
                                               (u0 + wlog * u1) * mask_f)

                    @pl.when(half == 1)
                    def _h1():
                        @plsc.parallel_loop(0, 128, unroll=8)
                        def _c1(cu):
                            cuv = jnp.broadcast_to(cu, (16,))
                            u2 = plsc.load_gather(table_v, [a2, cuv])
                            u3 = plsc.load_gather(table_v, [a3, cuv])
                            plsc.store_scatter(bufy_v, [cuv, cl],
                                               (u2 + wlog * u3) * mask_f)

                    return cc
                lax.fori_loop(0, 8, subchunk, 0)

                @pl.when(half == 0)
                def _dx():
                    pltpu.async_copy(
                        bufx_v.at[pl.ds(0, 128)],
                        outf.at[bj, m, pl.ds(296, 128), pl.ds(k * 128, 128)],
                        sem.at[0])

                @pl.when(half == 1)
                def _dy():
                    pltpu.async_copy(
                        bufy_v.at[pl.ds(0, 128)],
                        outf.at[bj, m, pl.ds(424, 128), pl.ds(k * 128, 128)],
                        sem.at[1])
                return bj

            lax.fori_loop(0, 2 * _BJOBS, unit, b0)
            pltpu.make_async_copy(
                bufx_v.at[pl.ds(0, 128)],
                outf.at[0, 0, pl.ds(296, 128), pl.ds(0, 128)],
                sem.at[0]).wait()
            pltpu.make_async_copy(
                bufy_v.at[pl.ds(0, 128)],
                outf.at[0, 0, pl.ds(424, 128), pl.ds(0, 128)],
                sem.at[1]).wait()

    return sc_kernel


_sc_call = _make_sc_call()


@jax.jit
def kernel(relation_path, path_info, graph_feature, context_feature,
           dis_embed, dis_sent_embed):
    rel1 = jnp.concatenate([
        relation_path.astype(jnp.int32).reshape(_NB * _NP * _NM * 4),
        jnp.zeros((_RPAD,), jnp.int32)])
    pinfo0 = path_info.astype(jnp.int32)[:, :, 0]
    gf = graph_feature.astype(jnp.float32)
    cf = context_feature.astype(jnp.float32)[:, :500, :]
    discat = jnp.concatenate(
        [dis_embed.astype(jnp.float32), dis_sent_embed.astype(jnp.float32)],
        axis=0)
    outf, outm = _sc_call(rel1, pinfo0, gf, cf, discat)
    path_fea = jnp.transpose(outf, (0, 3, 1, 2))[:, :_NP]
    mm = outm.reshape(_JOBS, 128).reshape(_NB, _NM, _NK * 128)[:, :, :_NP]
    mask = jnp.transpose(mm > 0, (0, 2, 1))
    return (path_fea, mask)


# speedup vs baseline: 1.0126x; 1.0126x over previous
import functools

import jax
import jax.numpy as jnp
from jax import lax
from jax.experimental import pallas as pl
from jax.experimental.pallas import tpu as pltpu
from jax.experimental.pallas import tpu_sc as plsc

_NB = 4
_NP = 1722
_NM = 12
_HID = 552
_NPP = 1792
_NK = 14
_JOBS = _NB * _NM * _NK
_NA = 12
_NBT = 20
_AJOBS = _JOBS // _NA
_BJOBS = -(-_JOBS // _NBT)
_JPB = _NM * _NK
_RPAD = (_NPP - _NP) * 48


def _bucket(x):
    e = lax.shift_right_logical(plsc.bitcast(x.astype(jnp.float32), jnp.int32), 23)
    return jnp.maximum(e - 126, 0)


def _make_sc_call():
    mesh = plsc.VectorSubcoreMesh(core_axis_name="c", subcore_axis_name="s")

    @functools.partial(
        pl.kernel,
        mesh=mesh,
        compiler_params=pltpu.CompilerParams(needs_layout_passes=False),
        out_type=[
            jax.ShapeDtypeStruct((_NB, _NM, _HID, _NPP), jnp.float32),
            jax.ShapeDtypeStruct((_NA, _AJOBS * 128), jnp.int32),
        ],
        scratch_types=[
            pltpu.VMEM((500, 128), jnp.float32),
            pltpu.VMEM((40, 20), jnp.float32),
            pltpu.VMEM((256,), jnp.int32),
            pltpu.VMEM((2 * 6144,), jnp.int32),
            pltpu.VMEM((144, 128), jnp.float32),
            pltpu.VMEM((152, 128), jnp.float32),
            pltpu.VMEM((_AJOBS * 128,), jnp.int32),
            pltpu.SemaphoreType.DMA((2,)),
            pltpu.SemaphoreType.DMA,
        ],
    )
    def sc_kernel(rel, pinfo, graph, ctx, discat, outf, outm,
                  table_v, discat_v, pinfo_v, idx_v, bufx_v, bufy_v,
                  mask_v, sem, sem2):
        cid = lax.axis_index("c")
        sid = lax.axis_index("s")
        wid = sid * 2 + cid
        lane = lax.iota(jnp.int32, 16)
        lane48 = lane * 48

        def slab_src(b, k):
            return rel.at[pl.ds((b * _NP + k * 128) * 48, 6144)]

        def load_ids(slot, m, cl):
            koff = slot * 6144 + cl * 48 + m * 4
            i0 = jnp.clip(plsc.load_gather(idx_v, [koff]), 0, 255)
            i1 = jnp.clip(plsc.load_gather(idx_v, [koff + 1]), 0, 255)
            i2 = jnp.clip(plsc.load_gather(idx_v, [koff + 2]), 0, 255)
            i3 = jnp.clip(plsc.load_gather(idx_v, [koff + 3]), 0, 255)
            return i0, i1, i2, i3

        @pl.when(wid < _NA)
        def _a_role():
            ta = wid
            b0 = ta // (_NA // _NB)
            pltpu.sync_copy(graph.at[b0], table_v.at[pl.ds(0, 256)])
            pltpu.sync_copy(discat, discat_v)
            pltpu.sync_copy(pinfo.at[b0], pinfo_v)
            r0 = lax.rem(ta * _AJOBS, _JPB)
            pltpu.sync_copy(slab_src(b0, lax.rem(r0, _NK)),
                            idx_v.at[pl.ds(0, 6144)])

            def unit(u, carry):
                i = u // 2
                half = lax.rem(u, 2)
                js = lax.rem(i, 2)
                r = lax.rem(ta * _AJOBS + i, _JPB)
                m = r // _NK
                k = lax.rem(r, _NK)

                @pl.when(jnp.logical_and(half == 0, i >= 1))
                def _ws():
                    pltpu.make_async_copy(
                        slab_src(b0, k),
                        idx_v.at[pl.ds(js * 6144, 6144)], sem2).wait()

                @pl.when(jnp.logical_and(half == 0, i + 1 < _AJOBS))
                def _ps():
                    r1 = lax.rem(ta * _AJOBS + i + 1, _JPB)
                    pltpu.async_copy(
                        slab_src(b0, lax.rem(r1, _NK)),
                        idx_v.at[pl.ds(lax.rem(i + 1, 2) * 6144, 6144)], sem2)

                @pl.when(jnp.logical_and(u >= 2, half == 0))
                def _wx():
                    pltpu.make_async_copy(
                        bufx_v,
                        outf.at[b0, 0, pl.ds(0, 144), pl.ds(0, 128)],
                        sem.at[0]).wait()

                @pl.when(jnp.logical_and(u >= 2, half == 1))
                def _wy():
                    pltpu.make_async_copy(
                        bufy_v,
                        outf.at[b0, 0, pl.ds(144, 152), pl.ds(0, 128)],
                        sem.at[1]).wait()

                def subchunk(c, cc):
                    cl = c * 16 + lane
                    i0, i1, i2, i3 = load_ids(js, m, cl)
                    isel = jnp.where(jnp.broadcast_to(m < 4, (16,)), i2, i3)
                    ssum = i0 + i1 + i2 + i3
                    mask_f = jnp.where(ssum > 0, 1.0, 0.0).astype(jnp.float32)

                    @pl.when(half == 0)
                    def _h0():
                        plsc.store_scatter(mask_v, [i * 128 + cl], ssum)

                        @plsc.parallel_loop(0, 128, unroll=4)
                        def _g0(cu):
                            cuv = jnp.broadcast_to(cu, (16,))
                            v = plsc.load_gather(table_v, [i0, cuv]) * mask_f
                            plsc.store_scatter(bufx_v, [cuv, cl], v)

                        @plsc.parallel_loop(0, 16, unroll=4)
                        def _g1(cu):
                            cuv = jnp.broadcast_to(cu, (16,))
                            v = plsc.load_gather(table_v, [isel, cuv]) * mask_f
                            plsc.store_scatter(bufx_v, [cuv + 128, cl], v)

                    @pl.when(half == 1)
                    def _h1():
                        a0 = plsc.load_gather(pinfo_v, [i0])
                        asel = plsc.load_gather(pinfo_v, [isel])
                        delta = a0 - asel
                        xeff = jnp.clip(
                            jnp.where(delta < 0, delta + 512, delta), 0, 511)
                        d = _bucket(xeff)
                        di = jnp.where(delta < 0, 10 - d, 10 + d)
                        di2 = _bucket(di) + 30

                        @plsc.parallel_loop(0, 112, unroll=4)
                        def _g2(cu):
                            cuv = jnp.broadcast_to(cu, (16,))
                            v = plsc.load_gather(
                                table_v, [isel, cuv + 16]) * mask_f
                            plsc.store_scatter(bufy_v, [cuv, cl], v)

                        @plsc.parallel_loop(0, 20, unroll=4)
                        def _g3(cu):
                            cuv = jnp.broadcast_to(cu, (16,))
                            e0 = plsc.load_gather(discat_v, [di, cuv]) * mask_f
                            plsc.store_scatter(bufy_v, [cuv + 112, cl], e0)
                            e1 = plsc.load_gather(discat_v, [di2, cuv]) * mask_f
                            plsc.store_scatter(bufy_v, [cuv + 132, cl], e1)

                    return cc
                lax.fori_loop(0, 8, subchunk, 0)

                @pl.when(half == 0)
                def _dx():
                    pltpu.async_copy(
                        bufx_v,
                        outf.at[b0, m, pl.ds(0, 144), pl.ds(k * 128, 128)],
                        sem.at[0])

                @pl.when(half == 1)
                def _dy():
                    pltpu.async_copy(
                        bufy_v,
                        outf.at[b0, m, pl.ds(144, 152), pl.ds(k * 128, 128)],
                        sem.at[1])
                return carry

            lax.fori_loop(0, 2 * _AJOBS, unit, 0)
            pltpu.make_async_copy(
                bufx_v, outf.at[b0, 0, pl.ds(0, 144), pl.ds(0, 128)],
                sem.at[0]).wait()
            pltpu.make_async_copy(
                bufy_v, outf.at[b0, 0, pl.ds(144, 152), pl.ds(0, 128)],
                sem.at[1]).wait()
            pltpu.sync_copy(mask_v, outm.at[ta])

        @pl.when(wid >= _NA)
        def _b_role():
            tb = wid - _NA
            b0 = (tb * _BJOBS) // _JPB
            pltpu.sync_copy(ctx.at[b0], table_v)
            pltpu.sync_copy(pinfo.at[b0], pinfo_v)
            r0 = lax.rem(tb * _BJOBS, _JPB)
            pltpu.sync_copy(slab_src(b0, lax.rem(r0, _NK)),
                            idx_v.at[pl.ds(0, 6144)])

            def unit(u, bcur):
                i = u // 2
                half = lax.rem(u, 2)
                js = lax.rem(i, 2)
                j = jnp.minimum(tb * _BJOBS + i, _JOBS - 1)
                bj = j // _JPB
                r = lax.rem(j, _JPB)
                m = r // _NK
                k = lax.rem(r, _NK)

                @pl.when(bj != bcur)
                def _reload():
                    pltpu.sync_copy(ctx.at[bj], table_v)
                    pltpu.sync_copy(pinfo.at[bj], pinfo_v)

                @pl.when(jnp.logical_and(half == 0, i >= 1))
                def _ws():
                    pltpu.make_async_copy(
                        slab_src(bj, k),
                        idx_v.at[pl.ds(js * 6144, 6144)], sem2).wait()

                @pl.when(jnp.logical_and(half == 0, i + 1 < _BJOBS))
                def _ps():
                    j1 = jnp.minimum(tb * _BJOBS + i + 1, _JOBS - 1)
                    r1 = lax.rem(j1, _JPB)
                    pltpu.async_copy(
                        slab_src(j1 // _JPB, lax.rem(r1, _NK)),
                        idx_v.at[pl.ds(lax.rem(i + 1, 2) * 6144, 6144)], sem2)

                @pl.when(jnp.logical_and(u >= 2, half == 0))
                def _wx():
                    pltpu.make_async_copy(
                        bufx_v.at[pl.ds(0, 128)],
                        outf.at[bj, 0, pl.ds(296, 128), pl.ds(0, 128)],
                        sem.at[0]).wait()

                @pl.when(jnp.logical_and(u >= 2, half == 1))
                def _wy():
                    pltpu.make_async_copy(
                        bufy_v.at[pl.ds(0, 128)],
                        outf.at[bj, 0, pl.ds(424, 128), pl.ds(0, 128)],
                        sem.at[1]).wait()

                def subchunk(c, cc):
                    cl = c * 16 + lane
                    i0, i1, i2, i3 = load_ids(js, m, cl)
                    ssum = i0 + i1 + i2 + i3
                    mask_f = jnp.where(ssum > 0, 1.0, 0.0).astype(jnp.float32)
                    wlog = jnp.where(jnp.broadcast_to(m >= 8, (16,)),
                                     1.0, 0.0).astype(jnp.float32)
                    a0 = jnp.clip(plsc.load_gather(pinfo_v, [i0]), 0, 499)
                    a1 = jnp.clip(plsc.load_gather(pinfo_v, [i1]), 0, 499)
                    a2 = jnp.clip(plsc.load_gather(pinfo_v, [i2]), 0, 499)
                    a3 = jnp.clip(plsc.load_gather(pinfo_v, [i3]), 0, 499)

                    @pl.when(half == 0)
                    def _h0():
                        @plsc.parallel_loop(0, 128, unroll=4)
                        def _c0(cu):
                            cuv = jnp.broadcast_to(cu, (16,))
                            u0 = plsc.load_gather(table_v, [a0, cuv])
                            u1 = plsc.load_gather(table_v, [a1, cuv])
                            plsc.store_scatter(bufx_v, [cuv, cl],
                                               (u0 + wlog * u1) * mask_f)

                    @pl.when(half == 1)
                    def _h1():
                        @plsc.parallel_loop(0, 128, unroll=4)
                        def _c1(cu):
                            cuv = jnp.broadcast_to(cu, (16,))
                            u2 = plsc.load_gather(table_v, [a2, cuv])
                            u3 = plsc.load_gather(table_v, [a3, cuv])
                            plsc.store_scatter(bufy_v, [cuv, cl],
                                               (u2 + wlog * u3) * mask_f)

                    return cc
                lax.fori_loop(0, 8, subchunk, 0)

                @pl.when(half == 0)
                def _dx():
                    pltpu.async_copy(
                        bufx_v.at[pl.ds(0, 128)],
                        outf.at[bj, m, pl.ds(296, 128), pl.ds(k * 128, 128)],
                        sem.at[0])

                @pl.when(half == 1)
                def _dy():
                    pltpu.async_copy(
                        bufy_v.at[pl.ds(0, 128)],
                        outf.at[bj, m, pl.ds(424, 128), pl.ds(k * 128, 128)],
                        sem.at[1])
                return bj

            lax.fori_loop(0, 2 * _BJOBS, unit, b0)
            pltpu.make_async_copy(
                bufx_v.at[pl.ds(0, 128)],
                outf.at[0, 0, pl.ds(296, 128), pl.ds(0, 128)],
                sem.at[0]).wait()
            pltpu.make_async_copy(
                bufy_v.at[pl.ds(0, 128)],
                outf.at[0, 0, pl.ds(424, 128), pl.ds(0, 128)],
                sem.at[1]).wait()

    return sc_kernel


_sc_call = _make_sc_call()


@jax.jit
def kernel(relation_path, path_info, graph_feature, context_feature,
           dis_embed, dis_sent_embed):
    rel1 = jnp.concatenate([
        relation_path.astype(jnp.int32).reshape(_NB * _NP * _NM * 4),
        jnp.zeros((_RPAD,), jnp.int32)])
    pinfo0 = path_info.astype(jnp.int32)[:, :, 0]
    gf = graph_feature.astype(jnp.float32)
    cf = context_feature.astype(jnp.float32)[:, :500, :]
    discat = jnp.concatenate(
        [dis_embed.astype(jnp.float32), dis_sent_embed.astype(jnp.float32)],
        axis=0)
    outf, outm = _sc_call(rel1, pinfo0, gf, cf, discat)
    path_fea = jnp.transpose(outf, (0, 3, 1, 2))[:, :_NP]
    mm = outm.reshape(_JOBS, 128).reshape(_NB, _NM, _NK * 128)[:, :, :_NP]
    mask = jnp.transpose(mm > 0, (0, 2, 1))
    return (path_fea, mask)
